# baseline (speedup 1.0000x reference)
import jax
import jax.numpy as jnp
from jax.experimental import pallas as pl
from jax.experimental.pallas import tpu as pltpu

_TILE = 128
_NG = 8


def _pairwise_body(len_ref, x_ref, t_ref, out_ref, s_ref, ls_ref, tm_ref):
    x = x_ref[:, :]
    ls = -(jnp.maximum(-x, 0.0) + jnp.log1p(jnp.exp(-jnp.abs(x))))
    ls_ref[:, :] = ls
    s_ref[:, :] = jnp.exp(ls)

    iota_j1 = jax.lax.broadcasted_iota(jnp.int32, (1, _TILE), 1)
    ones_row = jnp.ones((1, _TILE), jnp.float32)
    onef = jnp.float32(1.0)
    zerof = jnp.float32(0.0)
    nanf = jnp.float32(jnp.nan)

    def outer(v):
        return jax.lax.dot_general(
            v, ones_row, (((0,), (0,)), ((), ())),
            preferred_element_type=jnp.float32)

    def fold(v):
        w = v.reshape(16, 8, _TILE)
        w = w[:8] + w[8:]
        w = w[:4] + w[4:]
        w = w[:2] + w[2:]
        return w[0] + w[1]

    acc_a = jnp.zeros((8, _TILE), jnp.float32)
    cnt_a = jnp.zeros((8, _TILE), jnp.float32)
    acc_b = jnp.zeros((8, _TILE), jnp.float32)
    cnt_b = jnp.zeros((8, _TILE), jnp.float32)
    off = jnp.int32(0)
    for g in range(_NG):
        end = off + len_ref[g]
        lo = off // _TILE
        hi = (end + _TILE - 1) // _TILE
        off_g = off

        def mask_body(r, _, off=off_g, end=end):
            gi = iota_j1 + r * _TILE
            trow = t_ref[pl.ds(r, 1), :]
            tm_ref[pl.ds(r, 1), :] = jnp.where(
                (gi >= off) & (gi < end), trow, nanf)
            return 0
        jax.lax.fori_loop(lo, hi, mask_body, 0)

        def ti_body(ri, carry, lo=lo):
            acc1a, cnt1a, acc1b, cnt1b = carry
            si_row = s_ref[pl.ds(ri, 1), :]
            lsi_row = ls_ref[pl.ds(ri, 1), :]
            ti_row = tm_ref[pl.ds(ri, 1), :]

            s_i = outer(si_row)
            t_i = outer(ti_row)

            def tile(rj, live, acc2, cnt2):
                ls_i = outer(lsi_row)
                sj_row = s_ref[pl.ds(rj, 1), :]
                lsj_row = ls_ref[pl.ds(rj, 1), :]
                tj_row = jnp.where(live, tm_ref[pl.ds(rj, 1), :], nanf)
                p = jnp.log(s_i + sj_row)
                c1 = t_i > tj_row
                c2 = tj_row > t_i
                cf = jnp.where(c1 | c2, onef, zerof)
                lssel = jnp.where(c1, ls_i, jnp.where(c2, lsj_row, zerof))
                return acc2 + fold(cf * p - lssel), cnt2 + fold(cf)

            def diag_tile(rj, acc2, cnt2):
                ls_i = outer(lsi_row)
                sj_row = s_ref[pl.ds(rj, 1), :]
                tj_row = tm_ref[pl.ds(rj, 1), :]
                p = jnp.log(s_i + sj_row)
                c1 = t_i > tj_row
                contrib = jnp.where(c1, p - ls_i, zerof)
                return acc2 + fold(contrib), cnt2 + fold(
                    jnp.where(c1, onef, zerof))

            acc1a, cnt1a = diag_tile(ri, acc1a, cnt1a)

            def tj_body(k, carry2):
                acc2a, cnt2a, acc2b, cnt2b = carry2
                rj = lo + 4 * k
                acc2a, cnt2a = tile(rj, True, acc2a, cnt2a)
                acc2b, cnt2b = tile(rj + 1, rj + 1 < ri, acc2b, cnt2b)
                acc2a, cnt2a = tile(rj + 2, rj + 2 < ri, acc2a, cnt2a)
                acc2b, cnt2b = tile(rj + 3, rj + 3 < ri, acc2b, cnt2b)
                return acc2a, cnt2a, acc2b, cnt2b

            npairs = ri - lo
            return jax.lax.fori_loop(
                0, (npairs + 3) // 4, tj_body,
                (acc1a, cnt1a, acc1b, cnt1b))

        acc_a, cnt_a, acc_b, cnt_b = jax.lax.fori_loop(
            lo, hi, ti_body, (acc_a, cnt_a, acc_b, cnt_b))
        off = end

    total = jnp.sum(acc_a + acc_b)
    count = jnp.sum(cnt_a.astype(jnp.int32)) + jnp.sum(
        cnt_b.astype(jnp.int32))
    out_ref[0, 0] = jnp.where(
        count > 0, total / count.astype(jnp.float32), 0.0)


def kernel(logits, targets, lengths):
    x2d = logits.reshape(_TILE, _TILE)
    t2d = targets.reshape(_TILE, _TILE)
    out = pl.pallas_call(
        _pairwise_body,
        out_shape=jax.ShapeDtypeStruct((1, 1), jnp.float32),
        in_specs=[
            pl.BlockSpec(memory_space=pltpu.SMEM),
            pl.BlockSpec(memory_space=pltpu.VMEM),
            pl.BlockSpec(memory_space=pltpu.VMEM),
        ],
        out_specs=pl.BlockSpec(memory_space=pltpu.SMEM),
        scratch_shapes=[
            pltpu.VMEM((_TILE, _TILE), jnp.float32),
            pltpu.VMEM((_TILE, _TILE), jnp.float32),
            pltpu.VMEM((_TILE, _TILE), jnp.float32),
        ],
    )(lengths, x2d, t2d)
    return out[0, 0]


# speedup vs baseline: 1.0487x; 1.0487x over previous
import jax
import jax.numpy as jnp
from jax.experimental import pallas as pl
from jax.experimental.pallas import tpu as pltpu

_TILE = 128
_NG = 8


def _pairwise_body(len_ref, x_ref, t_ref, out_ref, s_ref, ls_ref, tm_ref):
    x = x_ref[:, :]
    ls = -(jnp.maximum(-x, 0.0) + jnp.log1p(jnp.exp(-jnp.abs(x))))
    ls_ref[:, :] = ls
    s_ref[:, :] = jnp.exp(ls)

    iota_j1 = jax.lax.broadcasted_iota(jnp.int32, (1, _TILE), 1)
    ones_row = jnp.ones((1, _TILE), jnp.float32)
    onef = jnp.float32(1.0)
    zerof = jnp.float32(0.0)
    nanf = jnp.float32(jnp.nan)

    def outer(v):
        return jax.lax.dot_general(
            v, ones_row, (((0,), (0,)), ((), ())),
            preferred_element_type=jnp.float32)

    def fold(v):
        w = v.reshape(16, 8, _TILE)
        w = w[:8] + w[8:]
        w = w[:4] + w[4:]
        w = w[:2] + w[2:]
        return w[0] + w[1]

    acc_a = jnp.zeros((8, _TILE), jnp.float32)
    cnt_a = jnp.zeros((8, _TILE), jnp.float32)
    acc_b = jnp.zeros((8, _TILE), jnp.float32)
    cnt_b = jnp.zeros((8, _TILE), jnp.float32)
    off = jnp.int32(0)
    for g in range(_NG):
        end = off + len_ref[g]
        lo = off // _TILE
        hi = (end + _TILE - 1) // _TILE
        off_g = off

        def mask_body(r, _, off=off_g, end=end):
            gi = iota_j1 + r * _TILE
            trow = t_ref[pl.ds(r, 1), :]
            tm_ref[pl.ds(r, 1), :] = jnp.where(
                (gi >= off) & (gi < end), trow, nanf)
            return 0
        jax.lax.fori_loop(lo, hi, mask_body, 0)

        def ti_body(ri, carry, lo=lo):
            acc1a, cnt1a, acc1b, cnt1b = carry
            si_row = s_ref[pl.ds(ri, 1), :]
            lsi_row = ls_ref[pl.ds(ri, 1), :]
            ti_row = tm_ref[pl.ds(ri, 1), :]

            s_i = outer(si_row)
            t_i = outer(ti_row)
            ls_i = outer(lsi_row)

            def tile(rj, live, acc2, cnt2):
                sj_row = s_ref[pl.ds(rj, 1), :]
                lsj_row = ls_ref[pl.ds(rj, 1), :]
                tj_row = jnp.where(live, tm_ref[pl.ds(rj, 1), :], nanf)
                p = jnp.log(s_i + sj_row)
                c1 = t_i > tj_row
                c2 = tj_row > t_i
                cf = jnp.where(c1 | c2, onef, zerof)
                lssel = jnp.where(c1, ls_i, jnp.where(c2, lsj_row, zerof))
                return acc2 + fold(cf * p - lssel), cnt2 + fold(cf)

            def diag_tile(rj, acc2, cnt2):
                sj_row = s_ref[pl.ds(rj, 1), :]
                tj_row = tm_ref[pl.ds(rj, 1), :]
                p = jnp.log(s_i + sj_row)
                c1 = t_i > tj_row
                contrib = jnp.where(c1, p - ls_i, zerof)
                return acc2 + fold(contrib), cnt2 + fold(
                    jnp.where(c1, onef, zerof))

            acc1a, cnt1a = diag_tile(ri, acc1a, cnt1a)

            def tj_body(k, carry2):
                acc2a, cnt2a, acc2b, cnt2b = carry2
                rj = lo + 2 * k
                acc2a, cnt2a = tile(rj, True, acc2a, cnt2a)
                acc2b, cnt2b = tile(rj + 1, rj + 1 < ri, acc2b, cnt2b)
                return acc2a, cnt2a, acc2b, cnt2b

            npairs = ri - lo
            return jax.lax.fori_loop(
                0, (npairs + 1) // 2, tj_body,
                (acc1a, cnt1a, acc1b, cnt1b))

        acc_a, cnt_a, acc_b, cnt_b = jax.lax.fori_loop(
            lo, hi, ti_body, (acc_a, cnt_a, acc_b, cnt_b))
        off = end

    total = jnp.sum(acc_a + acc_b)
    count = jnp.sum(cnt_a.astype(jnp.int32)) + jnp.sum(
        cnt_b.astype(jnp.int32))
    out_ref[0, 0] = jnp.where(
        count > 0, total / count.astype(jnp.float32), 0.0)


def kernel(logits, targets, lengths):
    x2d = logits.reshape(_TILE, _TILE)
    t2d = targets.reshape(_TILE, _TILE)
    out = pl.pallas_call(
        _pairwise_body,
        out_shape=jax.ShapeDtypeStruct((1, 1), jnp.float32),
        in_specs=[
            pl.BlockSpec(memory_space=pltpu.SMEM),
            pl.BlockSpec(memory_space=pltpu.VMEM),
            pl.BlockSpec(memory_space=pltpu.VMEM),
        ],
        out_specs=pl.BlockSpec(memory_space=pltpu.SMEM),
        scratch_shapes=[
            pltpu.VMEM((_TILE, _TILE), jnp.float32),
            pltpu.VMEM((_TILE, _TILE), jnp.float32),
            pltpu.VMEM((_TILE, _TILE), jnp.float32),
        ],
    )(lengths, x2d, t2d)
    return out[0, 0]
